# baseline (speedup 1.0000x reference)
import functools

import jax
import jax.numpy as jnp
from jax import lax
from jax.experimental import pallas as pl
from jax.experimental.pallas import tpu as pltpu
from jax.experimental.pallas import tpu_sc as plsc

DIM = 64
COL_BLOCK = 65536
IDX_CHUNK = 128
SC_COLS = 262144
CW = 512


def _proj_kernel(prompt_ref, wp_ref, bp_ref, wc_ref, out_ref):
    pe = (
        jnp.dot(prompt_ref[...], wp_ref[...], preferred_element_type=jnp.float32)
        + bp_ref[...]
    )
    nsq = jnp.sum(pe * pe)
    denom = jnp.maximum(jnp.sqrt(nsq), 1e-12)
    c = jnp.sum(pe * wc_ref[...]) / denom
    out_ref[...] = jnp.broadcast_to(c, (1, 16))


def _reduce_kernel(pt_ref, wc_ref, ones_ref, out_ref):
    x = pt_ref[...]
    wc_t = wc_ref[...]
    ones = ones_ref[...]
    dot = jax.lax.dot_general(
        wc_t, x, (((1,), (0,)), ((), ())), preferred_element_type=jnp.float32
    )
    nsq = jax.lax.dot_general(
        ones, x * x, (((1,), (0,)), ((), ())),
        preferred_element_type=jnp.float32,
    )
    denom = jnp.maximum(jnp.sqrt(nsq), 1e-12)
    out_ref[...] = (dot / denom).reshape(-1)


def _fast_rsqrt(x):
    i = plsc.bitcast(x, jnp.int32)
    i = 0x5F3759DF - lax.shift_right_logical(i, 1)
    y = plsc.bitcast(i, jnp.float32)
    for _ in range(3):
        y = y * (1.5 - 0.5 * x * y * y)
    return y


def _make_sc_reduce(num_models, tc_cols, num_workers):
    cpt = SC_COLS // num_workers
    n_chunks = cpt // CW
    mesh = plsc.VectorSubcoreMesh(core_axis_name="c", subcore_axis_name="s")

    @functools.partial(
        pl.kernel,
        mesh=mesh,
        out_type=jax.ShapeDtypeStruct((SC_COLS,), jnp.float32),
        compiler_params=pltpu.CompilerParams(
            needs_layout_passes=False, use_tc_tiling_on_sc=True
        ),
        scratch_types=[
            pltpu.VMEM((DIM, CW), jnp.float32),
            pltpu.VMEM((DIM,), jnp.float32),
            pltpu.VMEM((cpt,), jnp.float32),
            pltpu.SemaphoreType.DMA,
        ],
    )
    def sc_reduce(pt_hbm, wc_hbm, out_hbm, buf_v, wc_v, out_v, sem):
        wid = lax.axis_index("s") * 2 + lax.axis_index("c")
        my_base = wid * cpt
        pltpu.sync_copy(wc_hbm, wc_v)
        wc_chunks = [wc_v[pl.ds(jj * 16, 16)] for jj in range(DIM // 16)]
        wcs = [wc_chunks[r // 16][r % 16] for r in range(DIM)]

        def chunk_body(k, carry):
            a = pl.multiple_of(my_base + k * CW, 128)
            pltpu.async_copy(
                pt_hbm.at[:, pl.ds(a, CW)], buf_v, sem
            ).wait()

            def cg_body(cg, carry2):
                accd = jnp.zeros((16,), jnp.float32)
                accn = jnp.zeros((16,), jnp.float32)
                for r in range(DIM):
                    v = buf_v[r, pl.ds(cg * 16, 16)]
                    accd = accd + v * wcs[r]
                    accn = accn + v * v
                out_v[pl.ds(k * CW + cg * 16, 16)] = accd * _fast_rsqrt(accn)
                return carry2

            lax.fori_loop(0, CW // 16, cg_body, 0)
            return carry

        lax.fori_loop(0, n_chunks, chunk_body, 0)
        pltpu.sync_copy(out_v, out_hbm.at[pl.ds(wid * cpt, cpt)])

    return sc_reduce


def _make_sc_gather(batch, tc_cols, num_workers, bpw):
    n_chunks = bpw // IDX_CHUNK
    mesh = plsc.VectorSubcoreMesh(core_axis_name="c", subcore_axis_name="s")

    @functools.partial(
        pl.kernel,
        mesh=mesh,
        out_type=jax.ShapeDtypeStruct((batch,), jnp.float32),
        compiler_params=pltpu.CompilerParams(
            needs_layout_passes=False, use_tc_tiling_on_sc=False
        ),
        scratch_types=[
            pltpu.VMEM((bpw,), jnp.int32),
            [pltpu.VMEM((IDX_CHUNK,), jnp.int32) for _ in range(n_chunks)],
            [pltpu.VMEM((IDX_CHUNK,), jnp.int32) for _ in range(n_chunks)],
            pltpu.VMEM((bpw,), jnp.float32),
            pltpu.VMEM((bpw,), jnp.float32),
            pltpu.VMEM((16,), jnp.float32),
            pltpu.VMEM((bpw,), jnp.float32),
            pltpu.SemaphoreType.DMA,
        ],
    )
    def sc_gather(ids_hbm, rtc_hbm, rsc_hbm, c_hbm, out_hbm,
                  ids_v, i1_vs, i2_vs, g1_v, g2_v, c_v, out_v, sem):
        wid = lax.axis_index("s") * 2 + lax.axis_index("c")
        base = wid * bpw
        pltpu.sync_copy(ids_hbm.at[pl.ds(base, bpw)], ids_v)
        pltpu.sync_copy(c_hbm, c_v)

        for k in range(n_chunks):
            for i in range(IDX_CHUNK // 16):
                v = ids_v[pl.ds((k * (IDX_CHUNK // 16) + i) * 16, 16)]
                i1_vs[k][pl.ds(i * 16, 16)] = v
                i2_vs[k][pl.ds(i * 16, 16)] = jnp.minimum(v, SC_COLS - 1)

        copies = []
        for k in range(n_chunks):
            copies.append(
                pltpu.async_copy(
                    rtc_hbm.at[i1_vs[k]],
                    g1_v.at[pl.ds(k * IDX_CHUNK, IDX_CHUNK)],
                    sem,
                )
            )
            copies.append(
                pltpu.async_copy(
                    rsc_hbm.at[i2_vs[k]],
                    g2_v.at[pl.ds(k * IDX_CHUNK, IDX_CHUNK)],
                    sem,
                )
            )
        for cp in copies:
            cp.wait()

        c_vec = c_v[...]

        def sel_body(i, carry):
            v = ids_v[pl.ds(i * 16, 16)]
            g1 = g1_v[pl.ds(i * 16, 16)]
            g2 = g2_v[pl.ds(i * 16, 16)]
            out_v[pl.ds(i * 16, 16)] = (
                jnp.where(v < SC_COLS, g2, g1) - c_vec
            )
            return carry

        lax.fori_loop(0, bpw // 16, sel_body, 0)
        pltpu.sync_copy(out_v, out_hbm.at[pl.ds(base, bpw)])

    return sc_gather


def kernel(model_ids, prompt_embed, P, Wp, bp, Wc):
    batch = model_ids.shape[0]
    num_models = P.shape[0]
    info = plsc.get_sparse_core_info()
    num_workers = info.num_cores * info.num_subcores
    bpw = batch // num_workers
    tc_cols = num_models - SC_COLS

    c_out = pl.pallas_call(
        _proj_kernel,
        out_shape=jax.ShapeDtypeStruct((1, 16), jnp.float32),
    )(prompt_embed, Wp, bp.reshape(1, DIM), Wc.reshape(1, DIM))

    pt = P.T
    wc_flat = Wc.reshape(DIM)

    skip = SC_COLS // COL_BLOCK
    n_blocks = pl.cdiv(num_models - SC_COLS, COL_BLOCK)
    r_tc = pl.pallas_call(
        _reduce_kernel,
        grid=(n_blocks,),
        in_specs=[
            pl.BlockSpec((DIM, COL_BLOCK), lambda i: (0, i + skip)),
            pl.BlockSpec((1, DIM), lambda i: (0, 0)),
            pl.BlockSpec((1, DIM), lambda i: (0, 0)),
        ],
        out_specs=pl.BlockSpec((COL_BLOCK,), lambda i: (i + skip,)),
        out_shape=jax.ShapeDtypeStruct((num_models,), jnp.float32),
    )(pt, Wc.reshape(1, DIM), jnp.ones((1, DIM), jnp.float32))

    r_sc = _make_sc_reduce(num_models, tc_cols, num_workers)(pt, wc_flat)

    ids = model_ids.astype(jnp.int32)
    out = _make_sc_gather(batch, tc_cols, num_workers, bpw)(
        ids, r_tc, r_sc, c_out.reshape(16)
    )
    return out


# speedup vs baseline: 1.1436x; 1.1436x over previous
import functools

import jax
import jax.numpy as jnp
from jax import lax
from jax.experimental import pallas as pl
from jax.experimental.pallas import tpu as pltpu
from jax.experimental.pallas import tpu_sc as plsc

DIM = 64
COL_BLOCK = 65536
IDX_CHUNK = 128
SC_COLS = 262144
CW = 512


def _proj_kernel(prompt_ref, wp_ref, bp_ref, wc_ref, out_ref):
    pe = (
        jnp.dot(prompt_ref[...], wp_ref[...], preferred_element_type=jnp.float32)
        + bp_ref[...]
    )
    nsq = jnp.sum(pe * pe)
    denom = jnp.maximum(jnp.sqrt(nsq), 1e-12)
    c = jnp.sum(pe * wc_ref[...]) / denom
    out_ref[...] = jnp.broadcast_to(c, (1, 16))


def _reduce_kernel(pt_ref, wc_ref, ones_ref, out_ref):
    x = pt_ref[...]
    wc_t = wc_ref[...]
    ones = ones_ref[...]
    dot = jax.lax.dot_general(
        wc_t, x, (((1,), (0,)), ((), ())), preferred_element_type=jnp.float32
    )
    nsq = jax.lax.dot_general(
        ones, x * x, (((1,), (0,)), ((), ())),
        preferred_element_type=jnp.float32,
    )
    denom = jnp.maximum(jnp.sqrt(nsq), 1e-12)
    out_ref[...] = (dot / denom).reshape(-1)


def _fast_rsqrt(x):
    i = plsc.bitcast(x, jnp.int32)
    i = 0x5F3759DF - lax.shift_right_logical(i, 1)
    y = plsc.bitcast(i, jnp.float32)
    for _ in range(3):
        y = y * (1.5 - 0.5 * x * y * y)
    return y


def _make_sc_reduce(num_models, tc_cols, num_workers):
    cpt = SC_COLS // num_workers
    n_chunks = cpt // CW
    mesh = plsc.VectorSubcoreMesh(core_axis_name="c", subcore_axis_name="s")

    @functools.partial(
        pl.kernel,
        mesh=mesh,
        out_type=jax.ShapeDtypeStruct((SC_COLS,), jnp.float32),
        compiler_params=pltpu.CompilerParams(
            needs_layout_passes=False, use_tc_tiling_on_sc=True
        ),
        scratch_types=[
            pltpu.VMEM((DIM, CW), jnp.float32),
            pltpu.VMEM((DIM,), jnp.float32),
            pltpu.VMEM((cpt,), jnp.float32),
            pltpu.SemaphoreType.DMA,
        ],
    )
    def sc_reduce(pt_hbm, wc_hbm, out_hbm, buf_v, wc_v, out_v, sem):
        wid = lax.axis_index("s") * 2 + lax.axis_index("c")
        my_base = wid * cpt
        pltpu.sync_copy(wc_hbm, wc_v)
        wc_chunks = [wc_v[pl.ds(jj * 16, 16)] for jj in range(DIM // 16)]
        wcs = [wc_chunks[r // 16][r % 16] for r in range(DIM)]

        def chunk_body(k, carry):
            a = pl.multiple_of(my_base + k * CW, 128)
            pltpu.async_copy(
                pt_hbm.at[:, pl.ds(a, CW)], buf_v, sem
            ).wait()

            def cg_body(cg, carry2):
                accd = [jnp.zeros((16,), jnp.float32) for _ in range(4)]
                accn = [jnp.zeros((16,), jnp.float32) for _ in range(4)]
                for r in range(DIM):
                    v = buf_v[r, pl.ds(cg * 16, 16)]
                    accd[r % 4] = accd[r % 4] + v * wcs[r]
                    accn[r % 4] = accn[r % 4] + v * v
                d = (accd[0] + accd[1]) + (accd[2] + accd[3])
                n = (accn[0] + accn[1]) + (accn[2] + accn[3])
                out_v[pl.ds(k * CW + cg * 16, 16)] = d * _fast_rsqrt(n)
                return carry2

            lax.fori_loop(0, CW // 16, cg_body, 0)
            return carry

        lax.fori_loop(0, n_chunks, chunk_body, 0)
        pltpu.sync_copy(out_v, out_hbm.at[pl.ds(wid * cpt, cpt)])

    return sc_reduce


def _make_sc_gather(batch, tc_cols, num_workers, bpw):
    n_chunks = bpw // IDX_CHUNK
    mesh = plsc.VectorSubcoreMesh(core_axis_name="c", subcore_axis_name="s")

    @functools.partial(
        pl.kernel,
        mesh=mesh,
        out_type=jax.ShapeDtypeStruct((batch,), jnp.float32),
        compiler_params=pltpu.CompilerParams(
            needs_layout_passes=False, use_tc_tiling_on_sc=False
        ),
        scratch_types=[
            pltpu.VMEM((bpw,), jnp.int32),
            [pltpu.VMEM((IDX_CHUNK,), jnp.int32) for _ in range(n_chunks)],
            [pltpu.VMEM((IDX_CHUNK,), jnp.int32) for _ in range(n_chunks)],
            pltpu.VMEM((bpw,), jnp.float32),
            pltpu.VMEM((bpw,), jnp.float32),
            pltpu.VMEM((16,), jnp.float32),
            pltpu.VMEM((bpw,), jnp.float32),
            pltpu.SemaphoreType.DMA,
        ],
    )
    def sc_gather(ids_hbm, rtc_hbm, rsc_hbm, c_hbm, out_hbm,
                  ids_v, i1_vs, i2_vs, g1_v, g2_v, c_v, out_v, sem):
        wid = lax.axis_index("s") * 2 + lax.axis_index("c")
        base = wid * bpw
        pltpu.sync_copy(ids_hbm.at[pl.ds(base, bpw)], ids_v)
        pltpu.sync_copy(c_hbm, c_v)

        for k in range(n_chunks):
            for i in range(IDX_CHUNK // 16):
                v = ids_v[pl.ds((k * (IDX_CHUNK // 16) + i) * 16, 16)]
                i1_vs[k][pl.ds(i * 16, 16)] = v
                i2_vs[k][pl.ds(i * 16, 16)] = lax.bitwise_and(v, SC_COLS - 1)

        copies = []
        for k in range(n_chunks):
            copies.append(
                pltpu.async_copy(
                    rtc_hbm.at[i1_vs[k]],
                    g1_v.at[pl.ds(k * IDX_CHUNK, IDX_CHUNK)],
                    sem,
                )
            )
            copies.append(
                pltpu.async_copy(
                    rsc_hbm.at[i2_vs[k]],
                    g2_v.at[pl.ds(k * IDX_CHUNK, IDX_CHUNK)],
                    sem,
                )
            )
        for cp in copies:
            cp.wait()

        c_vec = c_v[...]

        def sel_body(i, carry):
            v = ids_v[pl.ds(i * 16, 16)]
            g1 = g1_v[pl.ds(i * 16, 16)]
            g2 = g2_v[pl.ds(i * 16, 16)]
            out_v[pl.ds(i * 16, 16)] = (
                jnp.where(v < SC_COLS, g2, g1) - c_vec
            )
            return carry

        lax.fori_loop(0, bpw // 16, sel_body, 0)
        pltpu.sync_copy(out_v, out_hbm.at[pl.ds(base, bpw)])

    return sc_gather


def kernel(model_ids, prompt_embed, P, Wp, bp, Wc):
    batch = model_ids.shape[0]
    num_models = P.shape[0]
    info = plsc.get_sparse_core_info()
    num_workers = info.num_cores * info.num_subcores
    bpw = batch // num_workers
    tc_cols = num_models - SC_COLS

    c_out = pl.pallas_call(
        _proj_kernel,
        out_shape=jax.ShapeDtypeStruct((1, 16), jnp.float32),
    )(prompt_embed, Wp, bp.reshape(1, DIM), Wc.reshape(1, DIM))

    pt = P.T
    wc_flat = Wc.reshape(DIM)

    skip = SC_COLS // COL_BLOCK
    n_blocks = pl.cdiv(num_models - SC_COLS, COL_BLOCK)
    r_tc = pl.pallas_call(
        _reduce_kernel,
        grid=(n_blocks,),
        in_specs=[
            pl.BlockSpec((DIM, COL_BLOCK), lambda i: (0, i + skip)),
            pl.BlockSpec((1, DIM), lambda i: (0, 0)),
            pl.BlockSpec((1, DIM), lambda i: (0, 0)),
        ],
        out_specs=pl.BlockSpec((COL_BLOCK,), lambda i: (i + skip,)),
        out_shape=jax.ShapeDtypeStruct((num_models,), jnp.float32),
    )(pt, Wc.reshape(1, DIM), jnp.ones((1, DIM), jnp.float32))

    r_sc = _make_sc_reduce(num_models, tc_cols, num_workers)(pt, wc_flat)

    ids = model_ids.astype(jnp.int32)
    out = _make_sc_gather(batch, tc_cols, num_workers, bpw)(
        ids, r_tc, r_sc, c_out.reshape(16)
    )
    return out


# speedup vs baseline: 1.7132x; 1.4980x over previous
import functools

import jax
import jax.numpy as jnp
from jax import lax
from jax.experimental import pallas as pl
from jax.experimental.pallas import tpu as pltpu
from jax.experimental.pallas import tpu_sc as plsc

DIM = 64
COL_BLOCK = 65536
IDX_CHUNK = 128
SC_COLS = 131072
CW = 512


def _proj_kernel(prompt_ref, wp_ref, bp_ref, wc_ref, out_ref):
    pe = (
        jnp.dot(prompt_ref[...], wp_ref[...], preferred_element_type=jnp.float32)
        + bp_ref[...]
    )
    nsq = jnp.sum(pe * pe)
    denom = jnp.maximum(jnp.sqrt(nsq), 1e-12)
    c = jnp.sum(pe * wc_ref[...]) / denom
    out_ref[...] = jnp.broadcast_to(c, (1, 16))


def _reduce_kernel(pt_ref, wc_ref, ones_ref, out_ref):
    x = pt_ref[...]
    wc_t = wc_ref[...]
    ones = ones_ref[...]
    dot = jax.lax.dot_general(
        wc_t, x, (((1,), (0,)), ((), ())), preferred_element_type=jnp.float32
    )
    nsq = jax.lax.dot_general(
        ones, x * x, (((1,), (0,)), ((), ())),
        preferred_element_type=jnp.float32,
    )
    denom = jnp.maximum(jnp.sqrt(nsq), 1e-12)
    out_ref[...] = (dot / denom).reshape(-1)


def _fast_rsqrt(x):
    i = plsc.bitcast(x, jnp.int32)
    i = 0x5F3759DF - lax.shift_right_logical(i, 1)
    y = plsc.bitcast(i, jnp.float32)
    for _ in range(3):
        y = y * (1.5 - 0.5 * x * y * y)
    return y


def _make_sc_reduce(num_models, tc_cols, num_workers):
    cpt = SC_COLS // num_workers
    n_chunks = cpt // CW
    mesh = plsc.VectorSubcoreMesh(core_axis_name="c", subcore_axis_name="s")

    @functools.partial(
        pl.kernel,
        mesh=mesh,
        out_type=jax.ShapeDtypeStruct((SC_COLS,), jnp.float32),
        compiler_params=pltpu.CompilerParams(
            needs_layout_passes=False, use_tc_tiling_on_sc=True
        ),
        scratch_types=[
            pltpu.VMEM((DIM, CW), jnp.float32),
            pltpu.VMEM((DIM, CW), jnp.float32),
            pltpu.VMEM((DIM,), jnp.float32),
            pltpu.VMEM((cpt,), jnp.float32),
            pltpu.SemaphoreType.DMA,
            pltpu.SemaphoreType.DMA,
        ],
    )
    def sc_reduce(pt_hbm, wc_hbm, out_hbm, buf0_v, buf1_v, wc_v, out_v,
                  sem0, sem1):
        wid = lax.axis_index("s") * 2 + lax.axis_index("c")
        my_base = wid * cpt
        pltpu.sync_copy(wc_hbm, wc_v)
        wc_chunks = [wc_v[pl.ds(jj * 16, 16)] for jj in range(DIM // 16)]
        wcs = [wc_chunks[r // 16][r % 16] for r in range(DIM)]
        bufs = [buf0_v, buf1_v]
        sems = [sem0, sem1]

        def fire(k):
            a = pl.multiple_of(my_base + k * CW, 128)
            return pltpu.async_copy(
                pt_hbm.at[:, pl.ds(a, CW)], bufs[k % 2], sems[k % 2]
            )

        descs = [None, None]
        descs[0] = fire(0)
        for k in range(n_chunks):
            buf_v = bufs[k % 2]
            descs[k % 2].wait()
            if k + 1 < n_chunks:
                descs[(k + 1) % 2] = fire(k + 1)

            def cg_body(cg, carry2, k=k, buf_v=buf_v):
                accd = [jnp.zeros((16,), jnp.float32) for _ in range(4)]
                accn = [jnp.zeros((16,), jnp.float32) for _ in range(4)]
                for r in range(DIM):
                    v = buf_v[r, pl.ds(cg * 16, 16)]
                    accd[r % 4] = accd[r % 4] + v * wcs[r]
                    accn[r % 4] = accn[r % 4] + v * v
                d = (accd[0] + accd[1]) + (accd[2] + accd[3])
                n = (accn[0] + accn[1]) + (accn[2] + accn[3])
                out_v[pl.ds(k * CW + cg * 16, 16)] = d * _fast_rsqrt(n)
                return carry2

            lax.fori_loop(0, CW // 16, cg_body, 0)

        pltpu.sync_copy(out_v, out_hbm.at[pl.ds(wid * cpt, cpt)])

    return sc_reduce


def _make_sc_gather(batch, tc_cols, num_workers, bpw):
    n_chunks = bpw // IDX_CHUNK
    mesh = plsc.VectorSubcoreMesh(core_axis_name="c", subcore_axis_name="s")

    @functools.partial(
        pl.kernel,
        mesh=mesh,
        out_type=jax.ShapeDtypeStruct((batch,), jnp.float32),
        compiler_params=pltpu.CompilerParams(
            needs_layout_passes=False, use_tc_tiling_on_sc=False
        ),
        scratch_types=[
            pltpu.VMEM((bpw,), jnp.int32),
            [pltpu.VMEM((IDX_CHUNK,), jnp.int32) for _ in range(n_chunks)],
            [pltpu.VMEM((IDX_CHUNK,), jnp.int32) for _ in range(n_chunks)],
            pltpu.VMEM((bpw,), jnp.float32),
            pltpu.VMEM((bpw,), jnp.float32),
            pltpu.VMEM((16,), jnp.float32),
            pltpu.VMEM((bpw,), jnp.float32),
            pltpu.SemaphoreType.DMA,
        ],
    )
    def sc_gather(ids_hbm, rtc_hbm, rsc_hbm, c_hbm, out_hbm,
                  ids_v, i1_vs, i2_vs, g1_v, g2_v, c_v, out_v, sem):
        wid = lax.axis_index("s") * 2 + lax.axis_index("c")
        base = wid * bpw
        pltpu.sync_copy(ids_hbm.at[pl.ds(base, bpw)], ids_v)
        pltpu.sync_copy(c_hbm, c_v)

        for k in range(n_chunks):
            for i in range(IDX_CHUNK // 16):
                v = ids_v[pl.ds((k * (IDX_CHUNK // 16) + i) * 16, 16)]
                i1_vs[k][pl.ds(i * 16, 16)] = v
                i2_vs[k][pl.ds(i * 16, 16)] = lax.bitwise_and(v, SC_COLS - 1)

        copies = []
        for k in range(n_chunks):
            copies.append(
                pltpu.async_copy(
                    rtc_hbm.at[i1_vs[k]],
                    g1_v.at[pl.ds(k * IDX_CHUNK, IDX_CHUNK)],
                    sem,
                )
            )
            copies.append(
                pltpu.async_copy(
                    rsc_hbm.at[i2_vs[k]],
                    g2_v.at[pl.ds(k * IDX_CHUNK, IDX_CHUNK)],
                    sem,
                )
            )
        for cp in copies:
            cp.wait()

        c_vec = c_v[...]

        def sel_body(i, carry):
            v = ids_v[pl.ds(i * 16, 16)]
            g1 = g1_v[pl.ds(i * 16, 16)]
            g2 = g2_v[pl.ds(i * 16, 16)]
            out_v[pl.ds(i * 16, 16)] = (
                jnp.where(v < SC_COLS, g2, g1) - c_vec
            )
            return carry

        lax.fori_loop(0, bpw // 16, sel_body, 0)
        pltpu.sync_copy(out_v, out_hbm.at[pl.ds(base, bpw)])

    return sc_gather


def kernel(model_ids, prompt_embed, P, Wp, bp, Wc):
    batch = model_ids.shape[0]
    num_models = P.shape[0]
    info = plsc.get_sparse_core_info()
    num_workers = info.num_cores * info.num_subcores
    bpw = batch // num_workers
    tc_cols = num_models - SC_COLS

    c_out = pl.pallas_call(
        _proj_kernel,
        out_shape=jax.ShapeDtypeStruct((1, 16), jnp.float32),
    )(prompt_embed, Wp, bp.reshape(1, DIM), Wc.reshape(1, DIM))

    pt = P.T
    wc_flat = Wc.reshape(DIM)

    skip = SC_COLS // COL_BLOCK
    n_blocks = pl.cdiv(num_models - SC_COLS, COL_BLOCK)
    r_tc = pl.pallas_call(
        _reduce_kernel,
        grid=(n_blocks,),
        in_specs=[
            pl.BlockSpec((DIM, COL_BLOCK), lambda i: (0, i + skip)),
            pl.BlockSpec((1, DIM), lambda i: (0, 0)),
            pl.BlockSpec((1, DIM), lambda i: (0, 0)),
        ],
        out_specs=pl.BlockSpec((COL_BLOCK,), lambda i: (i + skip,)),
        out_shape=jax.ShapeDtypeStruct((num_models,), jnp.float32),
    )(pt, Wc.reshape(1, DIM), jnp.ones((1, DIM), jnp.float32))

    r_sc = _make_sc_reduce(num_models, tc_cols, num_workers)(pt, wc_flat)

    ids = model_ids.astype(jnp.int32)
    out = _make_sc_gather(batch, tc_cols, num_workers, bpw)(
        ids, r_tc, r_sc, c_out.reshape(16)
    )
    return out


# speedup vs baseline: 1.7285x; 1.0089x over previous
import functools

import jax
import jax.numpy as jnp
from jax import lax
from jax.experimental import pallas as pl
from jax.experimental.pallas import tpu as pltpu
from jax.experimental.pallas import tpu_sc as plsc

DIM = 64
COL_BLOCK = 65536
IDX_CHUNK = 128


def _proj_kernel(prompt_ref, wp_ref, bp_ref, wc_ref, out_ref):
    pe = (
        jnp.dot(prompt_ref[...], wp_ref[...], preferred_element_type=jnp.float32)
        + bp_ref[...]
    )
    nsq = jnp.sum(pe * pe)
    denom = jnp.maximum(jnp.sqrt(nsq), 1e-12)
    c = jnp.sum(pe * wc_ref[...]) / denom
    out_ref[...] = jnp.broadcast_to(c, (1, 16))


def _reduce_kernel(pt_ref, wc_ref, ones_ref, c_ref, out_ref):
    x = pt_ref[...]
    wc_t = wc_ref[...]
    ones = ones_ref[...]
    dot = jax.lax.dot_general(
        wc_t, x, (((1,), (0,)), ((), ())), preferred_element_type=jnp.float32
    )
    nsq = jax.lax.dot_general(
        ones, x * x, (((1,), (0,)), ((), ())),
        preferred_element_type=jnp.float32,
    )
    denom = jnp.maximum(jnp.sqrt(nsq), 1e-12)
    out_ref[...] = (dot / denom - c_ref[0, 0]).reshape(-1)


def _make_sc_kernel(batch, num_workers, bpw):
    n_chunks = bpw // IDX_CHUNK
    mesh = plsc.VectorSubcoreMesh(core_axis_name="c", subcore_axis_name="s")

    @functools.partial(
        pl.kernel,
        mesh=mesh,
        out_type=jax.ShapeDtypeStruct((batch,), jnp.float32),
        compiler_params=pltpu.CompilerParams(
            needs_layout_passes=False, use_tc_tiling_on_sc=False
        ),
        scratch_types=[
            [pltpu.VMEM((IDX_CHUNK,), jnp.int32) for _ in range(n_chunks)],
            pltpu.VMEM((bpw,), jnp.float32),
            pltpu.SemaphoreType.DMA,
        ],
    )
    def sc_kernel(ids_hbm, r_hbm, out_hbm, idx_vs, out_v, sem):
        wid = lax.axis_index("s") * 2 + lax.axis_index("c")
        base = wid * bpw
        for k in range(n_chunks):
            pltpu.sync_copy(
                ids_hbm.at[pl.ds(base + k * IDX_CHUNK, IDX_CHUNK)], idx_vs[k]
            )
        copies = []
        for k in range(n_chunks):
            copies.append(
                pltpu.async_copy(
                    r_hbm.at[idx_vs[k]],
                    out_v.at[pl.ds(k * IDX_CHUNK, IDX_CHUNK)],
                    sem,
                )
            )
        for cp in copies:
            cp.wait()
        pltpu.sync_copy(out_v, out_hbm.at[pl.ds(base, bpw)])

    return sc_kernel


def kernel(model_ids, prompt_embed, P, Wp, bp, Wc):
    batch = model_ids.shape[0]
    num_models = P.shape[0]
    info = plsc.get_sparse_core_info()
    num_workers = info.num_cores * info.num_subcores
    bpw = batch // num_workers

    c_out = pl.pallas_call(
        _proj_kernel,
        out_shape=jax.ShapeDtypeStruct((1, 16), jnp.float32),
    )(prompt_embed, Wp, bp.reshape(1, DIM), Wc.reshape(1, DIM))

    pt = P.T
    n_blocks = pl.cdiv(num_models, COL_BLOCK)
    r = pl.pallas_call(
        _reduce_kernel,
        grid=(n_blocks,),
        in_specs=[
            pl.BlockSpec((DIM, COL_BLOCK), lambda i: (0, i)),
            pl.BlockSpec((1, DIM), lambda i: (0, 0)),
            pl.BlockSpec((1, DIM), lambda i: (0, 0)),
            pl.BlockSpec((1, 16), lambda i: (0, 0)),
        ],
        out_specs=pl.BlockSpec((COL_BLOCK,), lambda i: (i,)),
        out_shape=jax.ShapeDtypeStruct((num_models,), jnp.float32),
    )(pt, Wc.reshape(1, DIM), jnp.ones((1, DIM), jnp.float32), c_out)

    ids = model_ids.astype(jnp.int32)
    out = _make_sc_kernel(batch, num_workers, bpw)(ids, r)
    return out
